# baseline (speedup 1.0000x reference)
import functools

import jax
import jax.numpy as jnp
from jax import lax
from jax.experimental import pallas as pl
from jax.experimental.pallas import tpu as pltpu
from jax.experimental.pallas import tpu_sc as plsc

N = 10000
E = 320000
NC = 2
NS = 16
L = 16
NW = NC * NS
EPW = E // NW
G = EPW // L
HR = 80
R = 2 * HR

_mesh = plsc.VectorSubcoreMesh(
    core_axis_name="c", subcore_axis_name="s", num_cores=NC, num_subcores=NS)
_sc_params = pltpu.CompilerParams(needs_layout_passes=False)


@functools.partial(
    pl.kernel,
    out_type=jax.ShapeDtypeStruct((NW * R, 128), jnp.float32),
    mesh=_mesh,
    scratch_types=[
        pltpu.VMEM((EPW,), jnp.int32),
        pltpu.VMEM((EPW,), jnp.int32),
        pltpu.VMEM((R, 128), jnp.float32),
    ],
    compiler_params=_sc_params,
)
def _deg_kernel(ei, out, src_v, dst_v, deg_v):
    wid = lax.axis_index("s") * NC + lax.axis_index("c")
    base = wid * EPW
    pltpu.sync_copy(ei.at[pl.ds(base, EPW)], src_v)
    pltpu.sync_copy(ei.at[pl.ds(E + base, EPW)], dst_v)

    zero = jnp.zeros((L,), jnp.float32)

    @plsc.parallel_loop(0, R, unroll=4)
    def _zero(r):
        for k in range(8):
            deg_v[r, pl.ds(k * L, L)] = zero

    ones = jnp.ones((L,), jnp.float32)
    hr = jnp.int32(HR)
    m7 = jnp.int32(127)

    @plsc.parallel_loop(0, G, unroll=5)
    def _scatter(g):
        off = g * L
        s = src_v[pl.ds(off, L)]
        d = dst_v[pl.ds(off, L)]
        plsc.addupdate_scatter(deg_v, [s >> 7, s & m7], ones)
        plsc.addupdate_scatter(deg_v, [(d >> 7) + hr, d & m7], ones)

    pltpu.sync_copy(deg_v, out.at[pl.ds(wid * R, R)])


def _rne_hi16(b):
    lsb = (b >> 16) & jnp.uint32(1)
    return (b + jnp.uint32(0x7FFF) + lsb) >> 16


def _mm(x_ref, wg_ref, cw_ref, we_ref, fw_ref, q_ref):
    wy = jnp.dot(wg_ref[...], cw_ref[...], preferred_element_type=jnp.float32,
                 precision=lax.Precision.HIGHEST)
    ws = jnp.dot(we_ref[...], fw_ref[...], preferred_element_type=jnp.float32,
                 precision=lax.Precision.HIGHEST)
    w2 = jnp.concatenate([wy, ws], axis=1)
    q2 = jnp.dot(x_ref[...], w2, preferred_element_type=jnp.float32)
    by = _rne_hi16(lax.bitcast_convert_type(q2[:, 0:1], jnp.uint32))
    bs = _rne_hi16(lax.bitcast_convert_type(q2[:, 1:2], jnp.uint32))
    q_ref[...] = lax.bitcast_convert_type((by << 16) | bs, jnp.int32)


_mm_call = pl.pallas_call(
    _mm,
    out_shape=jax.ShapeDtypeStruct((N, 1), jnp.int32),
)


def _norms(degp_ref, nrm_ref):
    deg = degp_ref[0:R, :]
    for i in range(1, NW):
        deg = deg + degp_ref[i * R:(i + 1) * R, :]
    nrm_ref[...] = jnp.where(
        deg > 0, 1.0 / jnp.sqrt(jnp.maximum(deg, 1.0)), 0.0)


_norm_call = pl.pallas_call(
    _norms,
    out_shape=jax.ShapeDtypeStruct((R, 128), jnp.float32),
)


@functools.partial(
    pl.kernel,
    out_type=jax.ShapeDtypeStruct((NW * R, 128), jnp.float32),
    mesh=_mesh,
    scratch_types=[
        pltpu.VMEM((EPW,), jnp.int32),
        pltpu.VMEM((EPW,), jnp.int32),
        pltpu.VMEM((N,), jnp.int32),
        pltpu.VMEM((HR, 128), jnp.float32),
        pltpu.VMEM((R, 128), jnp.float32),
    ],
    compiler_params=_sc_params,
)
def _edge_kernel(ei, q_hbm, nrm_hbm, out, src_v, dst_v, q_v, on_v, acc_v):
    wid = lax.axis_index("s") * NC + lax.axis_index("c")
    base = wid * EPW
    pltpu.sync_copy(ei.at[pl.ds(base, EPW)], src_v)
    pltpu.sync_copy(ei.at[pl.ds(E + base, EPW)], dst_v)
    pltpu.sync_copy(q_hbm, q_v)
    pltpu.sync_copy(nrm_hbm.at[pl.ds(0, HR)], on_v)

    zero = jnp.zeros((L,), jnp.float32)

    @plsc.parallel_loop(0, R, unroll=4)
    def _zero(r):
        for k in range(8):
            acc_v[r, pl.ds(k * L, L)] = zero

    hr = jnp.int32(HR)
    m7 = jnp.int32(127)
    hi_mask = jnp.full((L,), -65536, jnp.int32)
    sixteen = jnp.int32(16)

    @plsc.parallel_loop(0, G, unroll=10)
    def _scatter(g):
        off = g * L
        s = src_v[pl.ds(off, L)]
        d = dst_v[pl.ds(off, L)]
        on = plsc.load_gather(on_v, [s >> 7, s & m7])
        qg = plsc.load_gather(q_v, [s])
        gy = plsc.bitcast(qg & hi_mask, jnp.float32)
        gs = plsc.bitcast(qg << sixteen, jnp.float32)
        dr = d >> 7
        dc = d & m7
        plsc.addupdate_scatter(acc_v, [dr, dc], gy * on)
        plsc.addupdate_scatter(acc_v, [dr + hr, dc], gs * on)

    pltpu.sync_copy(acc_v, out.at[pl.ds(wid * R, R)])


def _finish(accp_ref, nrm_ref, bg_ref, cw_ref, cb_ref, be_ref, fw_ref, fb_ref,
            y2_ref):
    acc = accp_ref[0:R, :]
    for i in range(1, NW):
        acc = acc + accp_ref[i * R:(i + 1) * R, :]
    inorm = nrm_ref[HR:R, :]
    cy = jnp.dot(bg_ref[...], cw_ref[...], preferred_element_type=jnp.float32,
                 precision=lax.Precision.HIGHEST)[0, 0] + cb_ref[0, 0]
    cs = jnp.dot(be_ref[...], fw_ref[...], preferred_element_type=jnp.float32,
                 precision=lax.Precision.HIGHEST)[0, 0] + fb_ref[0, 0]
    y2_ref[...] = jnp.concatenate(
        [acc[0:HR, :] * inorm + cy, acc[HR:R, :] * inorm + cs], axis=0)


_fin_call = pl.pallas_call(
    _finish,
    out_shape=jax.ShapeDtypeStruct((R, 128), jnp.float32),
)


def kernel(x, edge_index, W_gnn, b_gnn, W_est, b_est, fc_est_w, fc_est_b,
           cls_w, cls_b):
    ei_flat = edge_index.reshape(2 * E)
    degp = _deg_kernel(ei_flat)
    q = _mm_call(x, W_gnn, cls_w, W_est, fc_est_w)
    nrm = _norm_call(degp)
    accp = _edge_kernel(ei_flat, q.reshape(N), nrm)
    y2 = _fin_call(accp, nrm,
                   b_gnn.reshape(1, 128), cls_w, cls_b.reshape(1, 1),
                   b_est.reshape(1, 128), fc_est_w, fc_est_b.reshape(1, 1))
    y = y2[0:HR, :].reshape(HR * 128)[0:N].reshape(N, 1)
    s = y2[HR:R, :].reshape(HR * 128)[0:N].reshape(N, 1)
    return (y, s)


# speedup vs baseline: 1.0699x; 1.0699x over previous
import functools

import jax
import jax.numpy as jnp
from jax import lax
from jax.experimental import pallas as pl
from jax.experimental.pallas import tpu as pltpu
from jax.experimental.pallas import tpu_sc as plsc

N = 10000
E = 320000
NC = 2
NS = 16
L = 16
NW = NC * NS
EPW = E // NW
G = EPW // L
HR = 80
R = 2 * HR

_mesh = plsc.VectorSubcoreMesh(
    core_axis_name="c", subcore_axis_name="s", num_cores=NC, num_subcores=NS)
_sc_params = pltpu.CompilerParams(needs_layout_passes=False)


@functools.partial(
    pl.kernel,
    out_type=jax.ShapeDtypeStruct((NW * R, 128), jnp.float32),
    mesh=_mesh,
    scratch_types=[
        pltpu.VMEM((EPW,), jnp.int32),
        pltpu.VMEM((EPW,), jnp.int32),
        pltpu.VMEM((R, 128), jnp.float32),
    ],
    compiler_params=_sc_params,
)
def _deg_kernel(ei, out, src_v, dst_v, deg_v):
    wid = lax.axis_index("s") * NC + lax.axis_index("c")
    base = wid * EPW
    pltpu.sync_copy(ei.at[pl.ds(base, EPW)], src_v)
    pltpu.sync_copy(ei.at[pl.ds(E + base, EPW)], dst_v)

    zero = jnp.zeros((L,), jnp.float32)

    @plsc.parallel_loop(0, R, unroll=4)
    def _zero(r):
        for k in range(8):
            deg_v[r, pl.ds(k * L, L)] = zero

    ones = jnp.ones((L,), jnp.float32)
    hr = jnp.int32(HR)
    m7 = jnp.int32(127)

    @plsc.parallel_loop(0, G, unroll=5)
    def _scatter(g):
        off = g * L
        s = src_v[pl.ds(off, L)]
        d = dst_v[pl.ds(off, L)]
        plsc.addupdate_scatter(deg_v, [s >> 7, s & m7], ones)
        plsc.addupdate_scatter(deg_v, [(d >> 7) + hr, d & m7], ones)

    pltpu.sync_copy(deg_v, out.at[pl.ds(wid * R, R)])


def _rne_hi16(b):
    lsb = (b >> 16) & jnp.uint32(1)
    return (b + jnp.uint32(0x7FFF) + lsb) >> 16


def _mm(x_ref, wg_ref, cw_ref, we_ref, fw_ref, q_ref):
    wy = jnp.dot(wg_ref[...], cw_ref[...], preferred_element_type=jnp.float32,
                 precision=lax.Precision.HIGHEST)
    ws = jnp.dot(we_ref[...], fw_ref[...], preferred_element_type=jnp.float32,
                 precision=lax.Precision.HIGHEST)
    w2 = jnp.concatenate([wy, ws], axis=1)
    q2 = jnp.dot(x_ref[...], w2, preferred_element_type=jnp.float32)
    by = _rne_hi16(lax.bitcast_convert_type(q2[:, 0:1], jnp.uint32))
    bs = _rne_hi16(lax.bitcast_convert_type(q2[:, 1:2], jnp.uint32))
    packed = lax.bitcast_convert_type((by << 16) | bs, jnp.int32)
    q_ref[...] = jnp.concatenate(
        [packed, jnp.zeros((HR * 128 - N, 1), jnp.int32)], axis=0)


_mm_call = pl.pallas_call(
    _mm,
    out_shape=jax.ShapeDtypeStruct((HR * 128, 1), jnp.int32),
)


def _norms(degp_ref, q_ref, u_ref, in_ref):
    deg = degp_ref[0:R, :]
    for i in range(1, NW):
        deg = deg + degp_ref[i * R:(i + 1) * R, :]
    nrm = jnp.where(deg > 0, 1.0 / jnp.sqrt(jnp.maximum(deg, 1.0)), 0.0)
    onorm = nrm[0:HR, :]
    in_ref[...] = nrm[HR:R, :]
    qb = lax.bitcast_convert_type(q_ref[...], jnp.uint32)
    uy = lax.bitcast_convert_type(qb & jnp.uint32(0xFFFF0000),
                                  jnp.float32) * onorm
    us = lax.bitcast_convert_type(qb << 16, jnp.float32) * onorm
    by = _rne_hi16(lax.bitcast_convert_type(uy, jnp.uint32))
    bs = _rne_hi16(lax.bitcast_convert_type(us, jnp.uint32))
    u_ref[...] = lax.bitcast_convert_type((by << 16) | bs, jnp.int32)


_norm_call = pl.pallas_call(
    _norms,
    out_shape=(
        jax.ShapeDtypeStruct((HR, 128), jnp.int32),
        jax.ShapeDtypeStruct((HR, 128), jnp.float32),
    ),
)


@functools.partial(
    pl.kernel,
    out_type=jax.ShapeDtypeStruct((NW * R, 128), jnp.float32),
    mesh=_mesh,
    scratch_types=[
        pltpu.VMEM((EPW,), jnp.int32),
        pltpu.VMEM((EPW,), jnp.int32),
        pltpu.VMEM((HR, 128), jnp.int32),
        pltpu.VMEM((R, 128), jnp.float32),
    ],
    compiler_params=_sc_params,
)
def _edge_kernel(ei, u_hbm, out, src_v, dst_v, u_v, acc_v):
    wid = lax.axis_index("s") * NC + lax.axis_index("c")
    base = wid * EPW
    pltpu.sync_copy(ei.at[pl.ds(base, EPW)], src_v)
    pltpu.sync_copy(ei.at[pl.ds(E + base, EPW)], dst_v)
    pltpu.sync_copy(u_hbm, u_v)

    zero = jnp.zeros((L,), jnp.float32)

    @plsc.parallel_loop(0, R, unroll=4)
    def _zero(r):
        for k in range(8):
            acc_v[r, pl.ds(k * L, L)] = zero

    hr = jnp.int32(HR)
    m7 = jnp.int32(127)
    hi_mask = jnp.full((L,), -65536, jnp.int32)
    sixteen = jnp.int32(16)

    @plsc.parallel_loop(0, G, unroll=10)
    def _scatter(g):
        off = g * L
        s = src_v[pl.ds(off, L)]
        d = dst_v[pl.ds(off, L)]
        ug = plsc.load_gather(u_v, [s >> 7, s & m7])
        gy = plsc.bitcast(ug & hi_mask, jnp.float32)
        gs = plsc.bitcast(ug << sixteen, jnp.float32)
        dr = d >> 7
        dc = d & m7
        plsc.addupdate_scatter(acc_v, [dr, dc], gy)
        plsc.addupdate_scatter(acc_v, [dr + hr, dc], gs)

    pltpu.sync_copy(acc_v, out.at[pl.ds(wid * R, R)])


def _finish(accp_ref, nrm_ref, bg_ref, cw_ref, cb_ref, be_ref, fw_ref, fb_ref,
            y2_ref):
    acc = accp_ref[0:R, :]
    for i in range(1, NW):
        acc = acc + accp_ref[i * R:(i + 1) * R, :]
    inorm = nrm_ref[...]
    cy = jnp.dot(bg_ref[...], cw_ref[...], preferred_element_type=jnp.float32,
                 precision=lax.Precision.HIGHEST)[0, 0] + cb_ref[0, 0]
    cs = jnp.dot(be_ref[...], fw_ref[...], preferred_element_type=jnp.float32,
                 precision=lax.Precision.HIGHEST)[0, 0] + fb_ref[0, 0]
    y2_ref[...] = jnp.concatenate(
        [acc[0:HR, :] * inorm + cy, acc[HR:R, :] * inorm + cs], axis=0)


_fin_call = pl.pallas_call(
    _finish,
    out_shape=jax.ShapeDtypeStruct((R, 128), jnp.float32),
)


def kernel(x, edge_index, W_gnn, b_gnn, W_est, b_est, fc_est_w, fc_est_b,
           cls_w, cls_b):
    ei_flat = edge_index.reshape(2 * E)
    degp = _deg_kernel(ei_flat)
    q = _mm_call(x, W_gnn, cls_w, W_est, fc_est_w)
    u, inorm = _norm_call(degp, q.reshape(HR, 128))
    accp = _edge_kernel(ei_flat, u)
    y2 = _fin_call(accp, inorm,
                   b_gnn.reshape(1, 128), cls_w, cls_b.reshape(1, 1),
                   b_est.reshape(1, 128), fc_est_w, fc_est_b.reshape(1, 1))
    y = y2[0:HR, :].reshape(HR * 128)[0:N].reshape(N, 1)
    s = y2[HR:R, :].reshape(HR * 128)[0:N].reshape(N, 1)
    return (y, s)
